# Initial kernel scaffold; baseline (speedup 1.0000x reference)
import jax
import jax.numpy as jnp
from jax.experimental import pallas as pl


def kernel(hidden_states, Wr, br, Wu, bu, Wd, bd, Wsu, bsu, Wsd, bsd):
    raise NotImplementedError("write your pallas kernel here")


# speedup vs baseline: 2.4138x; 2.4138x over previous
import functools

import jax
import jax.numpy as jnp
from jax.experimental import pallas as pl
from jax.experimental.pallas import tpu as pltpu


def _gelu(x):
    return x * 0.5 * (1.0 + jax.lax.erf(x * 0.7071067811865476))


def _moe_body(x_ref, Wr_ref, br_ref, Wu_ref, bu_ref, Wd_ref, bd_ref,
              Wsu_ref, bsu_ref, Wsd_ref, bsd_ref,
              out_ref, idx_ref, w_ref, *, E):
    e = pl.program_id(0)
    x = x_ref[...]

    @pl.when(e == 0)
    def _init():
        logits = jnp.dot(x, Wr_ref[...],
                         preferred_element_type=jnp.float32) + br_ref[...]
        m = jnp.max(logits, axis=-1, keepdims=True)
        sumexp = jnp.sum(jnp.exp(logits - m), axis=-1, keepdims=True)
        w_ref[...] = 1.0 / sumexp
        ii = jax.lax.broadcasted_iota(jnp.int32, logits.shape, 1)
        idx_ref[...] = jnp.min(jnp.where(logits >= m, ii, E),
                               axis=-1, keepdims=True)
        sh = jnp.dot(_gelu(jnp.dot(x, Wsu_ref[...],
                                   preferred_element_type=jnp.float32)
                           + bsu_ref[...]),
                     Wsd_ref[...], preferred_element_type=jnp.float32)
        out_ref[...] = x + sh + bsd_ref[...]

    up = (jnp.dot(x, Wu_ref[0], preferred_element_type=jnp.float32)
          + bu_ref[0, 0])
    down = (jnp.dot(_gelu(up), Wd_ref[0],
                    preferred_element_type=jnp.float32) + bd_ref[0, 0])
    gate = jnp.where(idx_ref[...] == e, w_ref[...], 0.0)
    out_ref[...] += down * gate


def kernel(hidden_states, Wr, br, Wu, bu, Wd, bd, Wsu, bsu, Wsd, bsd):
    B, S, D = hidden_states.shape
    E = Wr.shape[1]
    F = Wu.shape[2]
    x = hidden_states.reshape(S, D)
    br2 = br.reshape(1, E)
    bsu2 = bsu.reshape(1, F)
    bsd2 = bsd.reshape(1, D)
    bu3 = bu.reshape(E, 1, F)
    bd3 = bd.reshape(E, 1, D)

    const = lambda *bshape: pl.BlockSpec(bshape, lambda e: (0,) * len(bshape))
    out = pl.pallas_call(
        functools.partial(_moe_body, E=E),
        grid=(E,),
        in_specs=[
            const(S, D),
            const(D, E),
            const(1, E),
            pl.BlockSpec((1, D, F), lambda e: (e, 0, 0)),
            pl.BlockSpec((1, 1, F), lambda e: (e, 0, 0)),
            pl.BlockSpec((1, F, D), lambda e: (e, 0, 0)),
            pl.BlockSpec((1, 1, D), lambda e: (e, 0, 0)),
            const(D, F),
            const(1, F),
            const(F, D),
            const(1, D),
        ],
        out_specs=const(S, D),
        out_shape=jax.ShapeDtypeStruct((S, D), jnp.float32),
        scratch_shapes=[
            pltpu.VMEM((S, 1), jnp.int32),
            pltpu.VMEM((S, 1), jnp.float32),
        ],
    )(x, Wr, br2, Wu, bu3, Wd, bd3, Wsu, bsu2, Wsd, bsd2)
    return out.reshape(B, S, D)


# speedup vs baseline: 3.9134x; 1.6213x over previous
import functools

import jax
import jax.numpy as jnp
from jax import lax
from jax.experimental import pallas as pl
from jax.experimental.pallas import tpu as pltpu
from jax.experimental.pallas import tpu_sc as plsc

T = 128


def _gelu(x):
    return x * 0.5 * (1.0 + lax.erf(x * 0.7071067811865476))


def _router_body(x_ref, Wr_ref, br_ref, Wsu_ref, bsu_ref, Wsd_ref, bsd_ref,
                 base_ref, p_ref, w16_ref, te_ref, *, E, NT):
    x = x_ref[...]
    S = x.shape[0]
    f32 = jnp.float32

    logits = jnp.dot(x, Wr_ref[...], preferred_element_type=f32) + br_ref[...]
    m = jnp.max(logits, axis=-1, keepdims=True)
    sumexp = jnp.sum(jnp.exp(logits - m), axis=-1, keepdims=True)
    w = 1.0 / sumexp
    ii = lax.broadcasted_iota(jnp.int32, logits.shape, 1)
    idx = jnp.min(jnp.where(logits >= m, ii, E), axis=-1, keepdims=True)
    onehot = (ii == idx).astype(f32)

    L = (lax.broadcasted_iota(jnp.int32, (S, S), 1)
         < lax.broadcasted_iota(jnp.int32, (S, S), 0)).astype(f32)
    R = jnp.dot(L, onehot, preferred_element_type=f32)
    rank = jnp.sum(R * onehot, axis=-1, keepdims=True)

    counts = jnp.sum(onehot, axis=0, keepdims=True)
    pc = jnp.floor((counts + (T - 1)) / T) * T
    M = (lax.broadcasted_iota(jnp.int32, (E, E), 0)
         < lax.broadcasted_iota(jnp.int32, (E, E), 1)).astype(f32)
    off = jnp.dot(pc, M, preferred_element_type=f32)
    p = jnp.sum(onehot * off, axis=-1, keepdims=True) + rank
    p_ref[...] = p.astype(jnp.int32)
    w16_ref[...] = jnp.broadcast_to(w, (S, 128))

    ones_col = jnp.ones((S, 1), f32)
    counts_col = lax.dot_general(onehot, ones_col, (((0,), (0,)), ((), ())),
                                 preferred_element_type=f32)
    pc_col = jnp.floor((counts_col + (T - 1)) / T) * T
    off_col = lax.dot_general(M, pc_col, (((0,), (0,)), ((), ())),
                              preferred_element_type=f32)
    cend_col = off_col + pc_col
    it = (lax.broadcasted_iota(jnp.int32, (E, NT), 1) * T).astype(f32)
    te = jnp.sum((cend_col <= it).astype(f32), axis=0, keepdims=True)
    te_ref[...] = jnp.minimum(te, E - 1).astype(jnp.int32)

    sh = jnp.dot(_gelu(jnp.dot(x, Wsu_ref[...], preferred_element_type=f32)
                       + bsu_ref[...]),
                 Wsd_ref[...], preferred_element_type=f32)
    base_ref[...] = x + sh + bsd_ref[...]


def _run_router(x, Wr, br, Wsu, bsu, Wsd, bsd, *, NT):
    S, D = x.shape
    E = Wr.shape[1]
    F = Wsu.shape[1]
    const = lambda *bshape: pl.BlockSpec(bshape, lambda: (0,) * len(bshape))
    return pl.pallas_call(
        functools.partial(_router_body, E=E, NT=NT),
        in_specs=[const(S, D), const(D, E), const(1, E),
                  const(D, F), const(1, F), const(F, D), const(1, D)],
        out_specs=[const(S, D), const(S, 1), const(S, 128), const(1, NT)],
        out_shape=[
            jax.ShapeDtypeStruct((S, D), jnp.float32),
            jax.ShapeDtypeStruct((S, 1), jnp.int32),
            jax.ShapeDtypeStruct((S, 128), jnp.float32),
            jax.ShapeDtypeStruct((1, NT), jnp.int32),
        ],
    )(x, Wr, br.reshape(1, E), Wsu, bsu.reshape(1, F), Wsd, bsd.reshape(1, D))


def _run_dispatch(p, x, w16, *, NTT):
    S, D = x.shape
    info = plsc.get_sparse_core_info()
    NC, NS = info.num_cores, info.num_subcores
    NW = NC * NS
    CH = S // NW
    mesh = plsc.VectorSubcoreMesh(core_axis_name="c", subcore_axis_name="s")

    @functools.partial(
        pl.kernel, mesh=mesh,
        out_type=[jax.ShapeDtypeStruct((NTT, D), jnp.float32),
                  jax.ShapeDtypeStruct((NTT, 128), jnp.float32)],
        scratch_types=[pltpu.VMEM((CH,), jnp.int32),
                       pltpu.VMEM((CH, D), jnp.float32),
                       pltpu.VMEM((CH, 128), jnp.float32),
                       pltpu.SemaphoreType.DMA],
    )
    def dispatch(p_hbm, x_hbm, w_hbm, xs_hbm, ws_hbm, p_v, x_v, w_v, sem):
        wid = lax.axis_index("s") * NC + lax.axis_index("c")
        row0 = wid * CH
        pltpu.sync_copy(p_hbm.at[pl.ds(row0, CH)], p_v)
        pltpu.sync_copy(x_hbm.at[pl.ds(row0, CH)], x_v)
        pltpu.sync_copy(w_hbm.at[pl.ds(row0, CH)], w_v)
        pltpu.async_copy(x_v, xs_hbm.at[p_v], sem).wait()
        pltpu.async_copy(w_v, ws_hbm.at[p_v], sem).wait()

    return dispatch(p, x, w16)


def _expert_body(te_ref, xs_ref, ws_ref, Wu_ref, bu_ref, Wd_ref, bd_ref,
                 ys_ref):
    f32 = jnp.float32
    h = _gelu(jnp.dot(xs_ref[...], Wu_ref[0], preferred_element_type=f32)
              + bu_ref[0, 0])
    down = jnp.dot(h, Wd_ref[0], preferred_element_type=f32) + bd_ref[0, 0]
    ys_ref[...] = down * ws_ref[:, :1]


def _run_experts(te, xs, ws, Wu, bu, Wd, bd, *, NT):
    NTT, D = xs.shape
    E, _, F = Wu.shape
    grid_spec = pltpu.PrefetchScalarGridSpec(
        num_scalar_prefetch=1,
        grid=(NT,),
        in_specs=[
            pl.BlockSpec((T, D), lambda i, te: (i, 0)),
            pl.BlockSpec((T, 128), lambda i, te: (i, 0)),
            pl.BlockSpec((1, D, F), lambda i, te: (te[i], 0, 0)),
            pl.BlockSpec((1, 1, F), lambda i, te: (te[i], 0, 0)),
            pl.BlockSpec((1, F, D), lambda i, te: (te[i], 0, 0)),
            pl.BlockSpec((1, 1, D), lambda i, te: (te[i], 0, 0)),
        ],
        out_specs=pl.BlockSpec((T, D), lambda i, te: (i, 0)),
    )
    return pl.pallas_call(
        _expert_body,
        grid_spec=grid_spec,
        out_shape=jax.ShapeDtypeStruct((NTT, D), jnp.float32),
    )(te, xs, ws, Wu, bu.reshape(E, 1, F), Wd, bd.reshape(E, 1, D))


def _run_combine(p, ys, base):
    S, D = base.shape
    info = plsc.get_sparse_core_info()
    NC, NS, L16 = info.num_cores, info.num_subcores, info.num_lanes
    NW = NC * NS
    CH = S // NW
    mesh = plsc.VectorSubcoreMesh(core_axis_name="c", subcore_axis_name="s")

    @functools.partial(
        pl.kernel, mesh=mesh,
        out_type=jax.ShapeDtypeStruct((S, D), jnp.float32),
        scratch_types=[pltpu.VMEM((CH,), jnp.int32),
                       pltpu.VMEM((CH, D), jnp.float32),
                       pltpu.VMEM((CH, D), jnp.float32),
                       pltpu.SemaphoreType.DMA],
    )
    def combine(p_hbm, ys_hbm, base_hbm, out_hbm, p_v, y_v, b_v, sem):
        wid = lax.axis_index("s") * NC + lax.axis_index("c")
        row0 = wid * CH
        pltpu.sync_copy(p_hbm.at[pl.ds(row0, CH)], p_v)
        copy = pltpu.async_copy(ys_hbm.at[p_v], y_v, sem)
        pltpu.sync_copy(base_hbm.at[pl.ds(row0, CH)], b_v)
        copy.wait()

        def row_body(r, _):
            def col_body(c, _):
                sl = pl.ds(c * L16, L16)
                b_v[r, sl] = b_v[r, sl] + y_v[r, sl]
                return 0
            return lax.fori_loop(0, D // L16, col_body, 0)
        lax.fori_loop(0, CH, row_body, 0)
        pltpu.sync_copy(b_v, out_hbm.at[pl.ds(row0, CH)])

    return combine(p, ys, base)


def kernel(hidden_states, Wr, br, Wu, bu, Wd, bd, Wsu, bsu, Wsd, bsd):
    B, S, D = hidden_states.shape
    E = Wr.shape[1]
    NT = E + (S - E) // T + 1
    NTT = NT * T
    x = hidden_states.reshape(S, D)

    base, p2, w16, te2 = _run_router(x, Wr, br, Wsu, bsu, Wsd, bsd, NT=NT)
    p = p2.reshape(S)
    te = te2.reshape(NT)
    xs, ws = _run_dispatch(p, x, w16, NTT=NTT)
    ys = _run_experts(te, xs, ws, Wu, bu, Wd, bd, NT=NT)
    out = _run_combine(p, ys, base)
    return out.reshape(B, S, D)


# speedup vs baseline: 4.1893x; 1.0705x over previous
import functools

import jax
import jax.numpy as jnp
from jax import lax
from jax.experimental import pallas as pl
from jax.experimental.pallas import tpu as pltpu
from jax.experimental.pallas import tpu_sc as plsc

T = 128


def _gelu(x):
    return x * 0.5 * (1.0 + lax.erf(x * 0.7071067811865476))


def _router_body(x_ref, Wr_ref, br_ref, Wsu_ref, bsu_ref, Wsd_ref, bsd_ref,
                 base_ref, p_ref, w16_ref, te_ref, *, E, NT):
    x = x_ref[...]
    S = x.shape[0]
    f32 = jnp.float32

    logits = jnp.dot(x, Wr_ref[...], preferred_element_type=f32) + br_ref[...]
    m = jnp.max(logits, axis=-1, keepdims=True)
    sumexp = jnp.sum(jnp.exp(logits - m), axis=-1, keepdims=True)
    w = 1.0 / sumexp
    ii = lax.broadcasted_iota(jnp.int32, logits.shape, 1)
    idx = jnp.min(jnp.where(logits >= m, ii, E), axis=-1, keepdims=True)
    onehot = (ii == idx).astype(f32)

    L = (lax.broadcasted_iota(jnp.int32, (S, S), 1)
         < lax.broadcasted_iota(jnp.int32, (S, S), 0)).astype(f32)
    R = jnp.dot(L, onehot, preferred_element_type=f32)
    rank = jnp.sum(R * onehot, axis=-1, keepdims=True)

    counts = jnp.sum(onehot, axis=0, keepdims=True)
    pc = jnp.floor((counts + (T - 1)) / T) * T
    M = (lax.broadcasted_iota(jnp.int32, (E, E), 0)
         < lax.broadcasted_iota(jnp.int32, (E, E), 1)).astype(f32)
    off = jnp.dot(pc, M, preferred_element_type=f32)
    p = jnp.sum(onehot * off, axis=-1, keepdims=True) + rank
    p_ref[...] = p.astype(jnp.int32)
    w16_ref[...] = jnp.broadcast_to(w, (S, 128))

    ones_col = jnp.ones((S, 1), f32)
    counts_col = lax.dot_general(onehot, ones_col, (((0,), (0,)), ((), ())),
                                 preferred_element_type=f32)
    pc_col = jnp.floor((counts_col + (T - 1)) / T) * T
    off_col = lax.dot_general(M, pc_col, (((0,), (0,)), ((), ())),
                              preferred_element_type=f32)
    cend_col = off_col + pc_col
    it = (lax.broadcasted_iota(jnp.int32, (E, NT), 1) * T).astype(f32)
    te = jnp.sum((cend_col <= it).astype(f32), axis=0, keepdims=True)
    te_ref[...] = jnp.minimum(te, E - 1).astype(jnp.int32)

    sh = jnp.dot(_gelu(jnp.dot(x, Wsu_ref[...], preferred_element_type=f32)
                       + bsu_ref[...]),
                 Wsd_ref[...], preferred_element_type=f32)
    base_ref[...] = x + sh + bsd_ref[...]


def _run_router(x, Wr, br, Wsu, bsu, Wsd, bsd, *, NT):
    S, D = x.shape
    E = Wr.shape[1]
    F = Wsu.shape[1]
    const = lambda *bshape: pl.BlockSpec(bshape, lambda: (0,) * len(bshape))
    return pl.pallas_call(
        functools.partial(_router_body, E=E, NT=NT),
        in_specs=[const(S, D), const(D, E), const(1, E),
                  const(D, F), const(1, F), const(F, D), const(1, D)],
        out_specs=[const(S, D), const(S, 1), const(S, 128), const(1, NT)],
        out_shape=[
            jax.ShapeDtypeStruct((S, D), jnp.float32),
            jax.ShapeDtypeStruct((S, 1), jnp.int32),
            jax.ShapeDtypeStruct((S, 128), jnp.float32),
            jax.ShapeDtypeStruct((1, NT), jnp.int32),
        ],
    )(x, Wr, br.reshape(1, E), Wsu, bsu.reshape(1, F), Wsd, bsd.reshape(1, D))


def _run_dispatch(p, x, w16, *, NTT):
    S, D = x.shape
    info = plsc.get_sparse_core_info()
    NC, NS = info.num_cores, info.num_subcores
    NW = NC * NS
    CH = S // NW
    mesh = plsc.VectorSubcoreMesh(core_axis_name="c", subcore_axis_name="s")

    @functools.partial(
        pl.kernel, mesh=mesh,
        out_type=[jax.ShapeDtypeStruct((NTT, D), jnp.float32),
                  jax.ShapeDtypeStruct((NTT, 128), jnp.float32)],
        scratch_types=[pltpu.VMEM((CH,), jnp.int32),
                       pltpu.VMEM((CH, D), jnp.float32),
                       pltpu.VMEM((CH, 128), jnp.float32),
                       pltpu.SemaphoreType.DMA],
    )
    def dispatch(p_hbm, x_hbm, w_hbm, xs_hbm, ws_hbm, p_v, x_v, w_v, sem):
        wid = lax.axis_index("s") * NC + lax.axis_index("c")
        row0 = wid * CH
        pltpu.sync_copy(p_hbm.at[pl.ds(row0, CH)], p_v)
        pltpu.sync_copy(x_hbm.at[pl.ds(row0, CH)], x_v)
        pltpu.sync_copy(w_hbm.at[pl.ds(row0, CH)], w_v)
        c1 = pltpu.async_copy(x_v, xs_hbm.at[p_v], sem)
        c2 = pltpu.async_copy(w_v, ws_hbm.at[p_v], sem)
        c1.wait()
        c2.wait()

    return dispatch(p, x, w16)


def _expert_body(te_ref, xs_ref, ws_ref, Wu_ref, bu_ref, Wd_ref, bd_ref,
                 ys_ref):
    f32 = jnp.float32
    h = _gelu(jnp.dot(xs_ref[...], Wu_ref[0], preferred_element_type=f32)
              + bu_ref[0, 0])
    down = jnp.dot(h, Wd_ref[0], preferred_element_type=f32) + bd_ref[0, 0]
    ys_ref[...] = down * ws_ref[:, :1]


def _run_experts(te, xs, ws, Wu, bu, Wd, bd, *, NT):
    NTT, D = xs.shape
    E, _, F = Wu.shape
    grid_spec = pltpu.PrefetchScalarGridSpec(
        num_scalar_prefetch=1,
        grid=(NT,),
        in_specs=[
            pl.BlockSpec((T, D), lambda i, te: (i, 0)),
            pl.BlockSpec((T, 128), lambda i, te: (i, 0)),
            pl.BlockSpec((1, D, F), lambda i, te: (te[i], 0, 0)),
            pl.BlockSpec((1, 1, F), lambda i, te: (te[i], 0, 0)),
            pl.BlockSpec((1, F, D), lambda i, te: (te[i], 0, 0)),
            pl.BlockSpec((1, 1, D), lambda i, te: (te[i], 0, 0)),
        ],
        out_specs=pl.BlockSpec((T, D), lambda i, te: (i, 0)),
    )
    return pl.pallas_call(
        _expert_body,
        grid_spec=grid_spec,
        out_shape=jax.ShapeDtypeStruct((NTT, D), jnp.float32),
    )(te, xs, ws, Wu, bu.reshape(E, 1, F), Wd, bd.reshape(E, 1, D))


def _run_combine(p, ys, base):
    S, D = base.shape
    info = plsc.get_sparse_core_info()
    NC, NS, L16 = info.num_cores, info.num_subcores, info.num_lanes
    NW = NC * NS
    CH = S // NW
    mesh = plsc.VectorSubcoreMesh(core_axis_name="c", subcore_axis_name="s")

    @functools.partial(
        pl.kernel, mesh=mesh,
        out_type=jax.ShapeDtypeStruct((S, D), jnp.float32),
        scratch_types=[pltpu.VMEM((CH,), jnp.int32),
                       pltpu.VMEM((CH, D), jnp.float32),
                       pltpu.VMEM((CH, D), jnp.float32),
                       pltpu.SemaphoreType.DMA],
    )
    def combine(p_hbm, ys_hbm, base_hbm, out_hbm, p_v, y_v, b_v, sem):
        wid = lax.axis_index("s") * NC + lax.axis_index("c")
        row0 = wid * CH
        pltpu.sync_copy(p_hbm.at[pl.ds(row0, CH)], p_v)
        copy = pltpu.async_copy(ys_hbm.at[p_v], y_v, sem)
        pltpu.sync_copy(base_hbm.at[pl.ds(row0, CH)], b_v)
        copy.wait()

        def row_body(r, _):
            for c in range(D // L16):
                sl = pl.ds(c * L16, L16)
                b_v[r, sl] = b_v[r, sl] + y_v[r, sl]
            return 0
        lax.fori_loop(0, CH, row_body, 0)
        pltpu.sync_copy(b_v, out_hbm.at[pl.ds(row0, CH)])

    return combine(p, ys, base)


def kernel(hidden_states, Wr, br, Wu, bu, Wd, bd, Wsu, bsu, Wsd, bsd):
    B, S, D = hidden_states.shape
    E = Wr.shape[1]
    NT = E + (S - E) // T + 1
    NTT = NT * T
    x = hidden_states.reshape(S, D)

    base, p2, w16, te2 = _run_router(x, Wr, br, Wsu, bsu, Wsd, bsd, NT=NT)
    p = p2.reshape(S)
    te = te2.reshape(NT)
    xs, ws = _run_dispatch(p, x, w16, NTT=NTT)
    ys = _run_experts(te, xs, ws, Wu, bu, Wd, bd, NT=NT)
    out = _run_combine(p, ys, base)
    return out.reshape(B, S, D)
